# baseline (speedup 1.0000x reference)
import functools
import jax
import jax.numpy as jnp
from jax import lax
from jax.experimental import pallas as pl
from jax.experimental.pallas import tpu as pltpu
from jax.experimental.pallas import tpu_sc as plsc

_B = 16384
_F = 16
_NW = 32
_BPW = _B // _NW
_CH = 128
_NCH = _BPW // _CH


def _svdpp_body(uidx_hbm, iidx_hbm, ut_hbm, it_hbm, ubt_hbm, ibt_hbm, gb_hbm,
                out_hbm,
                uidx_v, iidx_v, urows_v, irows_v, ub_v, ib_v, out_v, gb_v,
                sem):
    wid = lax.axis_index("s") * 2 + lax.axis_index("c")
    base = wid * _BPW

    for c in range(_NCH):
        pltpu.sync_copy(uidx_hbm.at[pl.ds(base + c * _CH, _CH)], uidx_v.at[c])
        pltpu.sync_copy(iidx_hbm.at[pl.ds(base + c * _CH, _CH)], iidx_v.at[c])
    pltpu.sync_copy(gb_hbm, gb_v)

    copies = []
    for c in range(_NCH):
        sl = pl.ds(c * _CH, _CH)
        copies.append(pltpu.async_copy(ut_hbm.at[uidx_v.at[c]], urows_v.at[sl], sem))
        copies.append(pltpu.async_copy(it_hbm.at[iidx_v.at[c]], irows_v.at[sl], sem))
        copies.append(pltpu.async_copy(ubt_hbm.at[uidx_v.at[c]], ub_v.at[sl], sem))
        copies.append(pltpu.async_copy(ibt_hbm.at[iidx_v.at[c]], ib_v.at[sl], sem))
    for cp in copies:
        cp.wait()

    gb = gb_v[...]
    lane = lax.iota(jnp.int32, 16)

    def block(j, carry):
        r0 = j * 16
        acc = gb
        for r in range(16):
            u = urows_v[r0 + r]
            v = irows_v[r0 + r]
            s = jnp.sum(u * v)
            acc = jnp.where(lane == r, acc + s, acc)
        sl = pl.ds(r0, 16)
        out_v[sl] = acc + ub_v[sl] + ib_v[sl]
        return carry

    lax.fori_loop(0, _BPW // 16, block, 0)
    pltpu.sync_copy(out_v, out_hbm.at[pl.ds(base, _BPW)])


def kernel(user_idx, item_idx, user_table, item_table, implicit_table,
           user_bias_table, item_bias_table, global_bias):
    del implicit_table
    uidx = user_idx.astype(jnp.int32)
    iidx = item_idx.astype(jnp.int32)
    ubt = user_bias_table.reshape(-1)
    ibt = item_bias_table.reshape(-1)
    gb16 = jnp.broadcast_to(global_bias, (16,)).astype(jnp.float32)

    mesh = plsc.VectorSubcoreMesh(core_axis_name="c", subcore_axis_name="s")
    run = pl.kernel(
        _svdpp_body,
        out_type=jax.ShapeDtypeStruct((_B,), jnp.float32),
        mesh=mesh,
        compiler_params=pltpu.CompilerParams(
            needs_layout_passes=False, use_tc_tiling_on_sc=False),
        scratch_types=[
            pltpu.VMEM((_NCH, _CH), jnp.int32),
            pltpu.VMEM((_NCH, _CH), jnp.int32),
            pltpu.VMEM((_BPW, _F), jnp.float32),
            pltpu.VMEM((_BPW, _F), jnp.float32),
            pltpu.VMEM((_BPW,), jnp.float32),
            pltpu.VMEM((_BPW,), jnp.float32),
            pltpu.VMEM((_BPW,), jnp.float32),
            pltpu.VMEM((16,), jnp.float32),
            pltpu.SemaphoreType.DMA,
        ],
    )
    return run(uidx, iidx, user_table, item_table, ubt, ibt, gb16)


# speedup vs baseline: 2.5006x; 2.5006x over previous
import functools
import jax
import jax.numpy as jnp
from jax import lax
from jax.experimental import pallas as pl
from jax.experimental.pallas import tpu as pltpu
from jax.experimental.pallas import tpu_sc as plsc

_B = 16384
_F = 16
_N = 1000000
_NW = 32
_BPW = _B // _NW
_W = 1024
_NFULL = 976
_RAG0 = _NFULL * _W
_RAGW = 512
_TAIL0 = _RAG0 + _RAGW
_NWIN = _NFULL + 1
_SENT = 1 << 30


def _svdpp_body(uidx_hbm, iidx_hbm, ut_hbm, it_hbm, utail_hbm, itail_hbm,
                ubt_hbm, ibt_hbm, gb_hbm,
                out_hbm,
                uidx_v, iidx_v, win_v, cu_i, cu_b, ci_i, ci_b,
                hw_i, hw_b, stage_v, sidx_v,
                ue_l, ve_l, ub_v, ib_v, out_v, gb_v, utail_v, itail_v,
                shu, shi,
                sem, sem_i):
    sid = lax.axis_index("s")
    cid = lax.axis_index("c")
    wid = sid * 2 + cid
    base = wid * _BPW
    t = sid
    wlo = (_NWIN * t) // 16
    whi = (_NWIN * (t + 1)) // 16
    whi_full = jnp.minimum(whi, _NFULL)
    lane = lax.iota(jnp.int32, 16)

    for c in range(_B // 2048):
        sl = pl.ds(c * 2048, 2048)
        pltpu.sync_copy(uidx_hbm.at[sl], uidx_v.at[sl])
        pltpu.sync_copy(iidx_hbm.at[sl], iidx_v.at[sl])
    pltpu.sync_copy(gb_hbm, gb_v)
    pltpu.sync_copy(utail_hbm, utail_v)
    pltpu.sync_copy(itail_hbm, itail_v)

    def cand(k, carry):
        nu, ni = carry
        sl = pl.ds(k * 16, 16)
        bvec = lane + k * 16
        uv = uidx_v[sl]
        wv = lax.shift_right_logical(uv, 10)
        mu = (wv >= wlo) & (wv < whi) & (uv < _TAIL0)
        plsc.store_compressed(cu_i.at[pl.ds(nu, 16)], uv, mask=mu)
        plsc.store_compressed(cu_b.at[pl.ds(nu, 16)], bvec, mask=mu)
        nu = nu + plsc.all_reduce_population_count(mu)[0]
        iv = iidx_v[sl]
        wv = lax.shift_right_logical(iv, 10)
        mi = (wv >= wlo) & (wv < whi) & (iv < _TAIL0)
        plsc.store_compressed(ci_i.at[pl.ds(ni, 16)], iv, mask=mi)
        plsc.store_compressed(ci_b.at[pl.ds(ni, 16)], bvec, mask=mi)
        ni = ni + plsc.all_reduce_population_count(mi)[0]
        return nu, ni

    nu, ni = lax.fori_loop(0, _B // 16, cand, (jnp.int32(0), jnp.int32(0)))
    sent = jnp.full((16,), _SENT, jnp.int32)
    cu_i[pl.ds(nu, 16)] = sent
    cu_b[pl.ds(nu, 16)] = jnp.zeros((16,), jnp.int32)
    ci_i[pl.ds(ni, 16)] = sent
    ci_b[pl.ds(ni, 16)] = jnp.zeros((16,), jnp.int32)

    def fire_u(w, buf):
        off = pl.multiple_of(w * _W, 128)
        return pltpu.async_copy(ut_hbm.at[:, pl.ds(off, _W)], win_v.at[buf],
                                sem)

    def fire_i(w, buf):
        off = pl.multiple_of(w * _W, 128)
        return pltpu.async_copy(it_hbm.at[:, pl.ds(off, _W)], win_v.at[buf],
                                sem_i)

    def wait_slab(buf):
        s = sem if buf == 0 else sem_i
        pltpu.make_async_copy(ut_hbm.at[:, pl.ds(0, _W)], win_v.at[buf],
                              s).wait()

    def process(w, wbase, wsize, buf, c_i, c_b, nc, sh):
        def scan_c(k, nh):
            sl = pl.ds(k * 16, 16)
            cv = c_i[sl]
            cb = c_b[sl]
            m = lax.shift_right_logical(cv, 10) == w
            plsc.store_compressed(hw_i.at[pl.ds(nh, 16)], cv, mask=m)
            plsc.store_compressed(hw_b.at[pl.ds(nh, 16)], cb, mask=m)
            return nh + plsc.all_reduce_population_count(m)[0]

        nh = lax.fori_loop(0, (nc + 15) // 16, scan_c, jnp.int32(0))
        hw_i[pl.ds(nh, 16)] = jnp.full((16,), wbase, jnp.int32)
        hw_b[pl.ds(nh, 16)] = jnp.full((16,), _B, jnp.int32)

        def hits(h, carry):
            hv = hw_i[pl.ds(h * 16, 16)]
            hb = hw_b[pl.ds(h * 16, 16)]
            col = jnp.minimum(hv - wbase, wsize - 1)
            for half in range(2):
                for r in range(8):
                    q = half * 8 + r
                    vec = plsc.load_gather(
                        win_v.at[buf], [lane, jnp.full((16,), col[q],
                                                       jnp.int32)])
                    stage_v[pl.ds(r * 16, 16)] = vec
                    sidx_v[pl.ds(r * 16, 16)] = hb[q] * 16 + lane
                pltpu.sync_copy(stage_v, sh.at[sidx_v])
            return carry

        lax.fori_loop(0, (nh + 15) // 16, hits, 0)

    @pl.when(wlo < whi_full)
    def _():
        fire_u(wlo, 0)
        fire_i(wlo, 1)

    def window2(k, carry):
        w = wlo + k
        wait_slab(0)
        process(w, w * _W, _W, 0, cu_i, cu_b, nu, shu)

        @pl.when(k + 1 < whi_full - wlo)
        def _():
            fire_u(w + 1, 0)

        wait_slab(1)
        process(w, w * _W, _W, 1, ci_i, ci_b, ni, shi)

        @pl.when(k + 1 < whi_full - wlo)
        def _():
            fire_i(w + 1, 1)

        return carry

    lax.fori_loop(0, whi_full - wlo, window2, 0)

    @pl.when(whi == _NWIN)
    def _():
        pltpu.sync_copy(ut_hbm.at[:, pl.ds(_RAG0, _RAGW)],
                        win_v.at[0, :, pl.ds(0, _RAGW)])
        process(jnp.int32(_NFULL), _RAG0, _RAGW, 0, cu_i, cu_b, nu, shu)
        pltpu.sync_copy(it_hbm.at[:, pl.ds(_RAG0, _RAGW)],
                        win_v.at[1, :, pl.ds(0, _RAGW)])
        process(jnp.int32(_NFULL), _RAG0, _RAGW, 1, ci_i, ci_b, ni, shi)

    plsc.subcore_barrier()

    pltpu.sync_copy(shu.at[pl.ds(base * 16, _BPW * 16)], ue_l)
    pltpu.sync_copy(shi.at[pl.ds(base * 16, _BPW * 16)], ve_l)
    bias_copies = []
    for c in range(_BPW // 128):
        sl = pl.ds(c * 128, 128)
        slg = pl.ds(base + c * 128, 128)
        bias_copies.append(
            pltpu.async_copy(ubt_hbm.at[uidx_v.at[slg]], ub_v.at[sl], sem))
        bias_copies.append(
            pltpu.async_copy(ibt_hbm.at[iidx_v.at[slg]], ib_v.at[sl], sem))
    for cp in bias_copies:
        cp.wait()
    gb = gb_v[...]

    def block(j, carry):
        r0 = j * 16
        uiv = uidx_v[pl.ds(base + r0, 16)]
        iiv = iidx_v[pl.ds(base + r0, 16)]
        acc = gb
        for r in range(16):
            u = ue_l[pl.ds((r0 + r) * 16, 16)]
            v = ve_l[pl.ds((r0 + r) * 16, 16)]
            ut_ix = jnp.maximum(uiv[r] - _TAIL0, 0) * 16 + lane
            it_ix = jnp.maximum(iiv[r] - _TAIL0, 0) * 16 + lane
            u = jnp.where(uiv[r] >= _TAIL0,
                          plsc.load_gather(utail_v, [ut_ix]), u)
            v = jnp.where(iiv[r] >= _TAIL0,
                          plsc.load_gather(itail_v, [it_ix]), v)
            s = jnp.sum(u * v)
            acc = jnp.where(lane == r, acc + s, acc)
        sl = pl.ds(r0, 16)
        out_v[sl] = acc + ub_v[sl] + ib_v[sl]
        return carry

    lax.fori_loop(0, _BPW // 16, block, 0)
    pltpu.sync_copy(out_v, out_hbm.at[pl.ds(base, _BPW)])


def kernel(user_idx, item_idx, user_table, item_table, implicit_table,
           user_bias_table, item_bias_table, global_bias):
    del implicit_table
    uidx = user_idx.astype(jnp.int32)
    iidx = item_idx.astype(jnp.int32)
    ut = user_table.T
    it = item_table.T
    utail = user_table[_TAIL0:].reshape(-1)
    itail = item_table[_TAIL0:].reshape(-1)
    ubt = user_bias_table.reshape(-1)
    ibt = item_bias_table.reshape(-1)
    gb16 = jnp.broadcast_to(global_bias, (16,)).astype(jnp.float32)

    mesh = plsc.VectorSubcoreMesh(core_axis_name="c", subcore_axis_name="s")
    run = pl.kernel(
        _svdpp_body,
        out_type=jax.ShapeDtypeStruct((_B,), jnp.float32),
        mesh=mesh,
        compiler_params=pltpu.CompilerParams(needs_layout_passes=False),
        scratch_types=[
            pltpu.VMEM((_B,), jnp.int32),
            pltpu.VMEM((_B,), jnp.int32),
            pltpu.VMEM((2, _F, _W), jnp.float32),
            pltpu.VMEM((1552,), jnp.int32),
            pltpu.VMEM((1552,), jnp.int32),
            pltpu.VMEM((1552,), jnp.int32),
            pltpu.VMEM((1552,), jnp.int32),
            pltpu.VMEM((528,), jnp.int32),
            pltpu.VMEM((528,), jnp.int32),
            pltpu.VMEM((128,), jnp.float32),
            pltpu.VMEM((128,), jnp.int32),
            pltpu.VMEM((_BPW * _F,), jnp.float32),
            pltpu.VMEM((_BPW * _F,), jnp.float32),
            pltpu.VMEM((_BPW,), jnp.float32),
            pltpu.VMEM((_BPW,), jnp.float32),
            pltpu.VMEM((_BPW,), jnp.float32),
            pltpu.VMEM((16,), jnp.float32),
            pltpu.VMEM((64 * _F,), jnp.float32),
            pltpu.VMEM((64 * _F,), jnp.float32),
            pltpu.VMEM_SHARED(((_B + 16) * _F,), jnp.float32),
            pltpu.VMEM_SHARED(((_B + 16) * _F,), jnp.float32),
            pltpu.SemaphoreType.DMA,
            pltpu.SemaphoreType.DMA,
        ],
    )
    return run(uidx, iidx, ut, it, utail, itail, ubt, ibt, gb16)
